# baseline (speedup 1.0000x reference)
import functools

import jax
import jax.numpy as jnp
from jax import lax
from jax.experimental import pallas as pl
from jax.experimental.pallas import tpu as pltpu
from jax.experimental.pallas import tpu_sc as plsc

S = 64
K = 8
R = 0.05
C = 64
P = 8192

ROWS_PER_W = 2
PIX_PER_W = ROWS_PER_W * S
CAP = 2048
NB = 8
BCAP = 512
COLS_PER_B = S // NB
R2 = R * R
BIG = 1e9
PT_CHUNKS = P // 16


def _kernel_body(pts_hbm, feat_hbm, img_hbm, depth_hbm,
                 pts_v, ci_v, bi_v,
                 bcnt_v, gidx_v, w_v, rows_v, out_v, depth_v, sem):
    wid = lax.axis_index("s") * 2 + lax.axis_index("c")
    iota = lax.iota(jnp.int32, 16)
    ones_m = iota < 16
    low8 = iota < 8
    iota3 = iota * 3

    pltpu.sync_copy(pts_hbm, pts_v)

    r0 = wid * ROWS_PER_W
    r0f = r0.astype(jnp.float32)
    y_lo = (r0f + 0.5) * (2.0 / S) - 1.0 - R
    y_hi = (r0f + (ROWS_PER_W - 1) + 0.5) * (2.0 / S) - 1.0 + R

    def scan_pts(i, cnt):
        ridx = i * 16 + iota
        yv = plsc.load_gather(pts_v, [i * 48 + iota3 + 1])
        m = (yv >= y_lo) & (yv <= y_hi)
        mi = m.astype(jnp.int32)
        pos = jnp.minimum(cnt + plsc.cumsum(mi) - 1, CAP - 1)
        plsc.store_scatter(ci_v, [pos], ridx, mask=m)
        return jnp.minimum(cnt + jnp.sum(mi), CAP)

    with jax.named_scope("ph_a_compact"):
        cnt = lax.fori_loop(0, PT_CHUNKS, scan_pts, jnp.int32(0))
    n_chunks = (cnt + 15) // 16

    def bin_one(b, _):
        bf = b.astype(jnp.float32)
        xlo = (bf * COLS_PER_B + 0.5) * (2.0 / S) - 1.0 - R
        xhi = (bf * COLS_PER_B + (COLS_PER_B - 1) + 0.5) * (2.0 / S) - 1.0 + R

        def bin_scan(j, bcnt):
            lane = j * 16 + iota
            vl = lane < cnt
            iv = jnp.where(vl, ci_v[pl.ds(j * 16, 16)], 0)
            xv = plsc.load_gather(pts_v, [iv * 3])
            m = vl & (xv >= xlo) & (xv <= xhi)
            mi = m.astype(jnp.int32)
            pos = b * BCAP + jnp.minimum(bcnt + plsc.cumsum(mi) - 1, BCAP - 1)
            plsc.store_scatter(bi_v, [pos], iv, mask=m)
            return jnp.minimum(bcnt + jnp.sum(mi), BCAP)

        bcnt = lax.fori_loop(0, n_chunks, bin_scan, jnp.int32(0))
        plsc.store_scatter(bcnt_v, [jnp.full((16,), b, jnp.int32)],
                           jnp.full((16,), bcnt, jnp.int32), mask=iota == 0)
        return _

    with jax.named_scope("ph_a2_bin"):
        lax.fori_loop(0, NB, bin_one, jnp.int32(0))
    bcv = bcnt_v[pl.ds(0, 16)]

    def per_col(col, _):
        b = col // COLS_PER_B
        cxp = (col.astype(jnp.float32) + 0.5) * (2.0 / S) - 1.0
        cyp0 = (r0.astype(jnp.float32) + 0.5) * (2.0 / S) - 1.0
        cyp1 = cyp0 + (2.0 / S)
        nb_b = jnp.sum(jnp.where(iota == b, bcv, 0))
        nbch = (nb_b + 15) // 16
        bin0 = b * BCAP

        def scan_chunk(j, carry):
            bz0, bp0, bz1, bp1 = carry
            rel = j * 16
            vl = rel + iota < nb_b
            iv = jnp.where(vl, bi_v[pl.ds(bin0 + rel, 16)], 0)
            iv3 = iv * 3
            xv = plsc.load_gather(pts_v, [iv3])
            yv = plsc.load_gather(pts_v, [iv3 + 1])
            zv = plsc.load_gather(pts_v, [iv3 + 2])
            dx = xv - cxp
            dxx = dx * dx
            dy0 = yv - cyp0
            dy1 = yv - cyp1
            d20 = dxx + dy0 * dy0
            d21 = dxx + dy1 * dy1
            zc0 = jnp.where(vl & (d20 < R2), zv, BIG)
            zc1 = jnp.where(vl & (d21 < R2), zv, BIG)
            pv = bin0 + rel + iota
            zd0, pd0 = plsc.sort_key_val(zc0, pv, descending=True)
            zd1, pd1 = plsc.sort_key_val(zc1, pv, descending=True)
            nz0, np0 = plsc.sort_key_val(jnp.where(low8, bz0, zd0),
                                         jnp.where(low8, bp0, pd0))
            nz1, np1 = plsc.sort_key_val(jnp.where(low8, bz1, zd1),
                                         jnp.where(low8, bp1, pd1))
            return (nz0, np0, nz1, np1)

        big0 = jnp.full((16,), BIG, jnp.float32)
        zero0 = jnp.zeros((16,), jnp.int32)
        bz0, bp0, bz1, bp1 = lax.fori_loop(
            0, nbch, scan_chunk, (big0, zero0, big0, zero0))

        for (bzv, bpv, ps, cyp) in ((bz0, bp0, col, cyp0),
                                    (bz1, bp1, col + S, cyp1)):
            valid = low8 & (bzv < 100.0)
            safe_p = jnp.where(valid, bpv, 0)
            gi = jnp.where(valid, plsc.load_gather(bi_v, [safe_p]), 0)
            gi3 = gi * 3
            gx = plsc.load_gather(pts_v, [gi3])
            gy = plsc.load_gather(pts_v, [gi3 + 1])
            gz = plsc.load_gather(pts_v, [gi3 + 2])
            dx = gx - cxp
            dy = gy - cyp
            d2 = dx * dx + dy * dy
            w = jnp.where(valid, 1.0 - d2 / jnp.float32(R2), 0.0)
            den = jnp.sum(w)
            denv = jnp.maximum(jnp.full((16,), den, jnp.float32), 1e-10)
            w = w / denv

            has0 = jnp.sum(jnp.where(valid & (iota == 0), 1, 0)) > 0
            z0 = jnp.sum(jnp.where(iota == 0, gz, 0.0))
            depth = jnp.where(has0, z0, -1.0)
            rowi = jnp.full((16,), ps // S, jnp.int32)
            coli = jnp.full((16,), col, jnp.int32)
            plsc.store_scatter(depth_v, [rowi, coli, jnp.zeros((16,), jnp.int32)],
                               jnp.full((16,), depth, jnp.float32),
                               mask=iota == 0)
            plsc.store_scatter(gidx_v, [ps * K + iota], gi, mask=low8)
            plsc.store_scatter(w_v, [ps * 16 + iota], w, mask=ones_m)
        return _

    with jax.named_scope("ph_b_topk"):
        lax.fori_loop(0, S, per_col, jnp.int32(0))

    copies = []
    for b in range(8):
        copies.append(pltpu.async_copy(
            feat_hbm.at[gidx_v.at[pl.ds(b * 128, 128)]],
            rows_v.at[pl.ds(b * 128, 128)], sem))
    for cp in copies:
        cp.wait()

    def composite(p, _):
        wv = w_v[pl.ds(p * 16, 16)]
        r = p // S
        c = p - r * S
        for cb in range(C // 16):
            acc = jnp.zeros((16,), jnp.float32)
            for k in range(K):
                wk = wv[k]
                acc = acc + wk * rows_v[p * K + k, pl.ds(cb * 16, 16)]
            out_v[r, c, pl.ds(cb * 16, 16)] = acc
        return _

    with jax.named_scope("ph_c_composite"):
        lax.fori_loop(0, PIX_PER_W, composite, jnp.int32(0))

    pltpu.sync_copy(out_v, img_hbm.at[0, pl.ds(r0, ROWS_PER_W)])
    pltpu.sync_copy(depth_v, depth_hbm.at[pl.ds(r0, ROWS_PER_W)])


@jax.jit
def kernel(points, features):
    mesh = plsc.VectorSubcoreMesh(core_axis_name="c", subcore_axis_name="s")
    run = functools.partial(
        pl.kernel,
        mesh=mesh,
        compiler_params=pltpu.CompilerParams(
            needs_layout_passes=False, use_tc_tiling_on_sc=False),
        out_type=[
            jax.ShapeDtypeStruct((1, S, S, C), jnp.float32),
            jax.ShapeDtypeStruct((S, S, 1), jnp.float32),
        ],
        scratch_types=[
            pltpu.VMEM((P * 3,), jnp.float32),
            pltpu.VMEM((CAP,), jnp.int32),
            pltpu.VMEM((NB * BCAP,), jnp.int32),
            pltpu.VMEM((16,), jnp.int32),
            pltpu.VMEM((PIX_PER_W * K,), jnp.int32),
            pltpu.VMEM((PIX_PER_W * 16,), jnp.float32),
            pltpu.VMEM((PIX_PER_W * K, C), jnp.float32),
            pltpu.VMEM((ROWS_PER_W, S, C), jnp.float32),
            pltpu.VMEM((ROWS_PER_W, S, 1), jnp.float32),
            pltpu.SemaphoreType.DMA,
        ],
    )(_kernel_body)

    images, depth = run(points.reshape(-1), features)
    return images, depth


# speedup vs baseline: 1.0644x; 1.0644x over previous
import functools

import jax
import jax.numpy as jnp
from jax import lax
from jax.experimental import pallas as pl
from jax.experimental.pallas import tpu as pltpu
from jax.experimental.pallas import tpu_sc as plsc

S = 64
K = 8
R = 0.05
C = 64
P = 8192

ROWS_PER_W = 2
PIX_PER_W = ROWS_PER_W * S
CAP = 2048
NB = 8
BCAP = 384
COLS_PER_B = S // NB
R2 = R * R
BIG = 1e9
PT_CHUNKS = P // 16


def _kernel_body(pts_hbm, feat_hbm, img_hbm, depth_hbm,
                 pts_v, ci_v, bx_v, by_v, bz_v, bi_v,
                 bcnt_v, gidx_v, w_v, rows_v, out_v, depth_v, sem):
    wid = lax.axis_index("s") * 2 + lax.axis_index("c")
    iota = lax.iota(jnp.int32, 16)
    ones_m = iota < 16
    low8 = iota < 8
    c0 = jnp.zeros((16,), jnp.int32)
    iota3 = iota * 3

    pltpu.sync_copy(pts_hbm, pts_v)

    r0 = wid * ROWS_PER_W
    r0f = r0.astype(jnp.float32)
    y_lo = (r0f + 0.5) * (2.0 / S) - 1.0 - R
    y_hi = (r0f + (ROWS_PER_W - 1) + 0.5) * (2.0 / S) - 1.0 + R

    def scan_pts(i, cnt):
        for h in range(2):
            ridx = i * 32 + h * 16 + iota
            yv = plsc.load_gather(pts_v, [(i * 2 + h) * 48 + iota3 + 1])
            m = (yv >= y_lo) & (yv <= y_hi)
            mi = m.astype(jnp.int32)
            pos = jnp.minimum(cnt + plsc.cumsum(mi) - 1, CAP - 1)
            plsc.store_scatter(ci_v, [pos], ridx, mask=m)
            cnt = jnp.minimum(cnt + jnp.sum(mi), CAP)
        return cnt

    with jax.named_scope("ph_a_compact"):
        cnt = lax.fori_loop(0, PT_CHUNKS // 2, scan_pts, jnp.int32(0))
    n_chunks = (cnt + 15) // 16

    def bin_one(b, _):
        bf = b.astype(jnp.float32)
        xlo = (bf * COLS_PER_B + 0.5) * (2.0 / S) - 1.0 - R
        xhi = (bf * COLS_PER_B + (COLS_PER_B - 1) + 0.5) * (2.0 / S) - 1.0 + R

        def bin_scan(j, bcnt):
            lane = j * 16 + iota
            vl = lane < cnt
            iv = jnp.where(vl, ci_v[pl.ds(j * 16, 16)], 0)
            iv3 = iv * 3
            xv = plsc.load_gather(pts_v, [iv3])
            m = vl & (xv >= xlo) & (xv <= xhi)
            mi = m.astype(jnp.int32)
            pos = b * BCAP + jnp.minimum(bcnt + plsc.cumsum(mi) - 1, BCAP - 1)
            yv = plsc.load_gather(pts_v, [iv3 + 1])
            zv = plsc.load_gather(pts_v, [iv3 + 2])
            plsc.store_scatter(bx_v, [pos], xv, mask=m)
            plsc.store_scatter(by_v, [pos], yv, mask=m)
            plsc.store_scatter(bz_v, [pos], zv, mask=m)
            plsc.store_scatter(bi_v, [pos], iv, mask=m)
            return jnp.minimum(bcnt + jnp.sum(mi), BCAP - 16)

        bcnt = lax.fori_loop(0, n_chunks, bin_scan, jnp.int32(0))
        plsc.store_scatter(bx_v, [b * BCAP + bcnt + iota],
                           jnp.full((16,), BIG, jnp.float32), mask=ones_m)
        plsc.store_scatter(bcnt_v, [jnp.full((16,), b, jnp.int32)],
                           jnp.full((16,), bcnt, jnp.int32), mask=iota == 0)
        return _

    with jax.named_scope("ph_a2_bin"):
        lax.fori_loop(0, NB, bin_one, jnp.int32(0))
    bcv = bcnt_v[pl.ds(0, 16)]

    cyp0 = (r0.astype(jnp.float32) + 0.5) * (2.0 / S) - 1.0
    cyp1 = cyp0 + (2.0 / S)

    def per_col(col, _):
        b = col // COLS_PER_B
        cxp = (col.astype(jnp.float32) + 0.5) * (2.0 / S) - 1.0
        nb_b = jnp.sum(jnp.where(iota == b, bcv, 0))
        nbch = (nb_b + 15) // 16
        bin0 = b * BCAP

        def scan_chunk(j, carry):
            bz0, bp0, bz1, bp1 = carry
            base = bin0 + j * 16
            xv = bx_v[pl.ds(base, 16)]
            yv = by_v[pl.ds(base, 16)]
            zv = bz_v[pl.ds(base, 16)]
            dx = xv - cxp
            dxx = dx * dx
            dy0 = yv - cyp0
            dy1 = yv - cyp1
            d20 = dxx + dy0 * dy0
            d21 = dxx + dy1 * dy1
            zc0 = jnp.where(d20 < R2, zv, BIG)
            zc1 = jnp.where(d21 < R2, zv, BIG)
            pv = base + iota
            zd0, pd0 = plsc.sort_key_val(zc0, pv, descending=True)
            zd1, pd1 = plsc.sort_key_val(zc1, pv, descending=True)
            nz0, np0 = plsc.sort_key_val(jnp.where(low8, bz0, zd0),
                                         jnp.where(low8, bp0, pd0))
            nz1, np1 = plsc.sort_key_val(jnp.where(low8, bz1, zd1),
                                         jnp.where(low8, bp1, pd1))
            return (nz0, np0, nz1, np1)

        big0 = jnp.full((16,), BIG, jnp.float32)
        zero0 = jnp.zeros((16,), jnp.int32)
        bz0, bp0, bz1, bp1 = lax.fori_loop(
            0, nbch, scan_chunk, (big0, zero0, big0, zero0))

        for (bzv, bpv, ps, cyp) in ((bz0, bp0, col, cyp0),
                                    (bz1, bp1, col + S, cyp1)):
            valid = low8 & (bzv < 100.0)
            safe_p = jnp.where(valid, bpv, 0)
            gi = jnp.where(valid, plsc.load_gather(bi_v, [safe_p]), 0)
            gx = plsc.load_gather(bx_v, [safe_p])
            gy = plsc.load_gather(by_v, [safe_p])
            gz = plsc.load_gather(bz_v, [safe_p])
            dx = gx - cxp
            dy = gy - cyp
            d2 = dx * dx + dy * dy
            w = jnp.where(valid, 1.0 - d2 / jnp.float32(R2), 0.0)
            den = jnp.sum(w)
            denv = jnp.maximum(jnp.full((16,), den, jnp.float32), 1e-10)
            w = w / denv

            has0 = jnp.sum(jnp.where(valid & (iota == 0), 1, 0)) > 0
            z0 = jnp.sum(jnp.where(iota == 0, gz, 0.0))
            depth = jnp.where(has0, z0, -1.0)
            rowi = jnp.full((16,), ps // S, jnp.int32)
            coli = jnp.full((16,), col, jnp.int32)
            plsc.store_scatter(depth_v, [rowi, coli, c0],
                               jnp.full((16,), depth, jnp.float32),
                               mask=iota == 0)
            plsc.store_scatter(gidx_v, [ps * K + iota], gi, mask=low8)
            plsc.store_scatter(w_v, [ps * 16 + iota], w, mask=ones_m)
        return _

    def bin_block(b, _):
        lo = b * COLS_PER_B
        lax.fori_loop(lo, lo + COLS_PER_B, per_col, jnp.int32(0))
        for segbase in (0, S * K):
            seg = segbase + b * (COLS_PER_B * K)
            pltpu.async_copy(
                feat_hbm.at[gidx_v.at[pl.ds(seg, COLS_PER_B * K)]],
                rows_v.at[pl.ds(seg, COLS_PER_B * K)], sem)
        return _

    with jax.named_scope("ph_b_topk"):
        lax.fori_loop(0, NB, bin_block, jnp.int32(0))

    for b in range(NB):
        for segbase in (0, S * K):
            seg = segbase + b * (COLS_PER_B * K)
            pltpu.make_async_copy(
                feat_hbm.at[gidx_v.at[pl.ds(seg, COLS_PER_B * K)]],
                rows_v.at[pl.ds(seg, COLS_PER_B * K)], sem).wait()

    def composite(p, _):
        wv = w_v[pl.ds(p * 16, 16)]
        r = p // S
        c = p - r * S
        for cb in range(C // 16):
            acc = jnp.zeros((16,), jnp.float32)
            for k in range(K):
                wk = wv[k]
                acc = acc + wk * rows_v[p * K + k, pl.ds(cb * 16, 16)]
            out_v[r, c, pl.ds(cb * 16, 16)] = acc
        return _

    with jax.named_scope("ph_c_composite"):
        lax.fori_loop(0, PIX_PER_W, composite, jnp.int32(0))

    pltpu.sync_copy(out_v, img_hbm.at[0, pl.ds(r0, ROWS_PER_W)])
    pltpu.sync_copy(depth_v, depth_hbm.at[pl.ds(r0, ROWS_PER_W)])


@jax.jit
def kernel(points, features):
    mesh = plsc.VectorSubcoreMesh(core_axis_name="c", subcore_axis_name="s")
    run = functools.partial(
        pl.kernel,
        mesh=mesh,
        compiler_params=pltpu.CompilerParams(
            needs_layout_passes=False, use_tc_tiling_on_sc=False),
        out_type=[
            jax.ShapeDtypeStruct((1, S, S, C), jnp.float32),
            jax.ShapeDtypeStruct((S, S, 1), jnp.float32),
        ],
        scratch_types=[
            pltpu.VMEM((P * 3,), jnp.float32),
            pltpu.VMEM((CAP,), jnp.int32),
            pltpu.VMEM((NB * BCAP,), jnp.float32),
            pltpu.VMEM((NB * BCAP,), jnp.float32),
            pltpu.VMEM((NB * BCAP,), jnp.float32),
            pltpu.VMEM((NB * BCAP,), jnp.int32),
            pltpu.VMEM((16,), jnp.int32),
            pltpu.VMEM((PIX_PER_W * K,), jnp.int32),
            pltpu.VMEM((PIX_PER_W * 16,), jnp.float32),
            pltpu.VMEM((PIX_PER_W * K, C), jnp.float32),
            pltpu.VMEM((ROWS_PER_W, S, C), jnp.float32),
            pltpu.VMEM((ROWS_PER_W, S, 1), jnp.float32),
            pltpu.SemaphoreType.DMA,
        ],
    )(_kernel_body)

    images, depth = run(points.reshape(-1), features)
    return images, depth


# speedup vs baseline: 1.1772x; 1.1059x over previous
import functools

import jax
import jax.numpy as jnp
from jax import lax
from jax.experimental import pallas as pl
from jax.experimental.pallas import tpu as pltpu
from jax.experimental.pallas import tpu_sc as plsc

S = 64
K = 8
R = 0.05
C = 64
P = 8192

ROWS_PER_W = 2
PIX_PER_W = ROWS_PER_W * S
CAP = 2048
NB = 8
BCAP = 384
COLS_PER_B = S // NB
R2 = R * R
BIG = 1e9
PT_CHUNKS = P // 16


def _kernel_body(pts_hbm, feat_hbm, img_hbm, depth_hbm,
                 pts_v, ci_v, bx_v, by_v, bz_v, bi_v,
                 bcnt_v, gidx_v, w_v, rows_v, out_v, depth_v, sem):
    wid = lax.axis_index("s") * 2 + lax.axis_index("c")
    iota = lax.iota(jnp.int32, 16)
    ones_m = iota < 16
    low8 = iota < 8

    pltpu.sync_copy(pts_hbm, pts_v)

    r0 = wid * ROWS_PER_W
    r0f = r0.astype(jnp.float32)
    y_lo = (r0f + 0.5) * (2.0 / S) - 1.0 - R
    y_hi = (r0f + (ROWS_PER_W - 1) + 0.5) * (2.0 / S) - 1.0 + R

    def scan_pts(i, cnt):
        for h in range(2):
            ridx = i * 32 + h * 16 + iota
            yv = pts_v[pl.ds(P + i * 32 + h * 16, 16)]
            m = (yv >= y_lo) & (yv <= y_hi)
            mi = m.astype(jnp.int32)
            pos = jnp.minimum(cnt + plsc.cumsum(mi) - 1, CAP - 1)
            plsc.store_scatter(ci_v, [pos], ridx, mask=m)
            cnt = jnp.minimum(cnt + jnp.sum(mi), CAP)
        return cnt

    with jax.named_scope("ph_a_compact"):
        cnt = lax.fori_loop(0, PT_CHUNKS // 2, scan_pts, jnp.int32(0))
    n_chunks = (cnt + 15) // 16

    def bin_one(b, _):
        bf = b.astype(jnp.float32)
        xlo = (bf * COLS_PER_B + 0.5) * (2.0 / S) - 1.0 - R
        xhi = (bf * COLS_PER_B + (COLS_PER_B - 1) + 0.5) * (2.0 / S) - 1.0 + R

        def bin_scan(j, bcnt):
            lane = j * 16 + iota
            vl = lane < cnt
            iv = jnp.where(vl, ci_v[pl.ds(j * 16, 16)], 0)
            xv = plsc.load_gather(pts_v, [iv])
            m = vl & (xv >= xlo) & (xv <= xhi)
            mi = m.astype(jnp.int32)
            pos = b * BCAP + jnp.minimum(bcnt + plsc.cumsum(mi) - 1, BCAP - 1)
            yv = plsc.load_gather(pts_v, [iv + P])
            zv = plsc.load_gather(pts_v, [iv + 2 * P])
            plsc.store_scatter(bx_v, [pos], xv, mask=m)
            plsc.store_scatter(by_v, [pos], yv, mask=m)
            plsc.store_scatter(bz_v, [pos], zv, mask=m)
            plsc.store_scatter(bi_v, [pos], iv, mask=m)
            return jnp.minimum(bcnt + jnp.sum(mi), BCAP - 16)

        bcnt = lax.fori_loop(0, n_chunks, bin_scan, jnp.int32(0))
        plsc.store_scatter(bx_v, [b * BCAP + bcnt + iota],
                           jnp.full((16,), BIG, jnp.float32), mask=ones_m)
        plsc.store_scatter(bcnt_v, [jnp.full((16,), b, jnp.int32)],
                           jnp.full((16,), bcnt, jnp.int32), mask=iota == 0)
        return _

    with jax.named_scope("ph_a2_bin"):
        lax.fori_loop(0, NB, bin_one, jnp.int32(0))
    bcv = bcnt_v[pl.ds(0, 16)]

    cyp0 = (r0.astype(jnp.float32) + 0.5) * (2.0 / S) - 1.0
    cyp1 = cyp0 + (2.0 / S)

    def per_col(col, _):
        b = col // COLS_PER_B
        cxp = (col.astype(jnp.float32) + 0.5) * (2.0 / S) - 1.0
        nb_b = jnp.sum(jnp.where(iota == b, bcv, 0))
        nbch = (nb_b + 15) // 16
        bin0 = b * BCAP

        def scan_chunk(j, carry):
            bz0, bp0, bz1, bp1 = carry
            base = bin0 + j * 16
            xv = bx_v[pl.ds(base, 16)]
            yv = by_v[pl.ds(base, 16)]
            zv = bz_v[pl.ds(base, 16)]
            dx = xv - cxp
            dxx = dx * dx
            dy0 = yv - cyp0
            dy1 = yv - cyp1
            d20 = dxx + dy0 * dy0
            d21 = dxx + dy1 * dy1
            zc0 = jnp.where(d20 < R2, zv, BIG)
            zc1 = jnp.where(d21 < R2, zv, BIG)
            pv = base + iota
            zd0, pd0 = plsc.sort_key_val(zc0, pv, descending=True)
            zd1, pd1 = plsc.sort_key_val(zc1, pv, descending=True)
            nz0, np0 = plsc.sort_key_val(jnp.where(low8, bz0, zd0),
                                         jnp.where(low8, bp0, pd0))
            nz1, np1 = plsc.sort_key_val(jnp.where(low8, bz1, zd1),
                                         jnp.where(low8, bp1, pd1))
            return (nz0, np0, nz1, np1)

        big0 = jnp.full((16,), BIG, jnp.float32)
        zero0 = jnp.zeros((16,), jnp.int32)
        bz0, bp0, bz1, bp1 = lax.fori_loop(
            0, nbch, scan_chunk, (big0, zero0, big0, zero0))

        for (bzv, bpv, ps, cyp) in ((bz0, bp0, col, cyp0),
                                    (bz1, bp1, col + S, cyp1)):
            valid = low8 & (bzv < 100.0)
            safe_p = jnp.where(valid, bpv, 0)
            gi = jnp.where(valid, plsc.load_gather(bi_v, [safe_p]), 0)
            gx = plsc.load_gather(bx_v, [safe_p])
            gy = plsc.load_gather(by_v, [safe_p])
            gz = plsc.load_gather(bz_v, [safe_p])
            dx = gx - cxp
            dy = gy - cyp
            d2 = dx * dx + dy * dy
            w = jnp.where(valid, 1.0 - d2 / jnp.float32(R2), 0.0)
            den = jnp.sum(w)
            denv = jnp.maximum(jnp.full((16,), den, jnp.float32), 1e-10)
            w = w / denv

            has0 = jnp.sum(jnp.where(valid & (iota == 0), 1, 0)) > 0
            z0 = jnp.sum(jnp.where(iota == 0, gz, 0.0))
            depth = jnp.where(has0, z0, -1.0)
            rowi = jnp.full((16,), ps // S, jnp.int32)
            coli = jnp.full((16,), col, jnp.int32)
            plsc.store_scatter(depth_v, [rowi, coli],
                               jnp.full((16,), depth, jnp.float32),
                               mask=iota == 0)
            plsc.store_scatter(gidx_v, [ps * K + iota], gi, mask=low8)
            plsc.store_scatter(w_v, [ps * 16 + iota], w, mask=ones_m)
        return _

    def bin_block(b, _):
        lo = b * COLS_PER_B
        lax.fori_loop(lo, lo + COLS_PER_B, per_col, jnp.int32(0))
        for segbase in (0, S * K):
            seg = segbase + b * (COLS_PER_B * K)
            pltpu.async_copy(
                feat_hbm.at[gidx_v.at[pl.ds(seg, COLS_PER_B * K)]],
                rows_v.at[pl.ds(seg, COLS_PER_B * K)], sem)
        return _

    with jax.named_scope("ph_b_topk"):
        lax.fori_loop(0, NB, bin_block, jnp.int32(0))

    for b in range(NB):
        for segbase in (0, S * K):
            seg = segbase + b * (COLS_PER_B * K)
            pltpu.make_async_copy(
                feat_hbm.at[gidx_v.at[pl.ds(seg, COLS_PER_B * K)]],
                rows_v.at[pl.ds(seg, COLS_PER_B * K)], sem).wait()

    def composite(p, _):
        wv = w_v[pl.ds(p * 16, 16)]
        r = p // S
        c = p - r * S
        for cb in range(C // 16):
            acc = jnp.zeros((16,), jnp.float32)
            for k in range(K):
                wk = wv[k]
                acc = acc + wk * rows_v[p * K + k, pl.ds(cb * 16, 16)]
            out_v[r, c, pl.ds(cb * 16, 16)] = acc
        return _

    with jax.named_scope("ph_c_composite"):
        lax.fori_loop(0, PIX_PER_W, composite, jnp.int32(0))

    pltpu.sync_copy(out_v, img_hbm.at[0, pl.ds(r0, ROWS_PER_W)])
    pltpu.sync_copy(depth_v, depth_hbm.at[pl.ds(r0, ROWS_PER_W)])


@jax.jit
def kernel(points, features):
    mesh = plsc.VectorSubcoreMesh(core_axis_name="c", subcore_axis_name="s")
    run = functools.partial(
        pl.kernel,
        mesh=mesh,
        compiler_params=pltpu.CompilerParams(
            needs_layout_passes=False, use_tc_tiling_on_sc=False),
        out_type=[
            jax.ShapeDtypeStruct((1, S, S, C), jnp.float32),
            jax.ShapeDtypeStruct((S, S), jnp.float32),
        ],
        scratch_types=[
            pltpu.VMEM((P * 3,), jnp.float32),
            pltpu.VMEM((CAP,), jnp.int32),
            pltpu.VMEM((NB * BCAP,), jnp.float32),
            pltpu.VMEM((NB * BCAP,), jnp.float32),
            pltpu.VMEM((NB * BCAP,), jnp.float32),
            pltpu.VMEM((NB * BCAP,), jnp.int32),
            pltpu.VMEM((16,), jnp.int32),
            pltpu.VMEM((PIX_PER_W * K,), jnp.int32),
            pltpu.VMEM((PIX_PER_W * 16,), jnp.float32),
            pltpu.VMEM((PIX_PER_W * K, C), jnp.float32),
            pltpu.VMEM((ROWS_PER_W, S, C), jnp.float32),
            pltpu.VMEM((ROWS_PER_W, S), jnp.float32),
            pltpu.SemaphoreType.DMA,
        ],
    )(_kernel_body)

    images, depth = run(points.T.reshape(-1), features)
    return images, depth[..., None]


# speedup vs baseline: 1.2462x; 1.0586x over previous
import functools

import jax
import jax.numpy as jnp
from jax import lax
from jax.experimental import pallas as pl
from jax.experimental.pallas import tpu as pltpu
from jax.experimental.pallas import tpu_sc as plsc

S = 64
K = 8
R = 0.05
C = 64
P = 8192

ROWS_PER_W = 2
PIX_PER_W = ROWS_PER_W * S
CAP = 2048
NB = 8
BCAP = 384
COLS_PER_B = S // NB
R2 = R * R
BIG = 1e9
PT_CHUNKS = P // 16


def _kernel_body(pts_hbm, feat_hbm, img_hbm, depth_hbm,
                 pts_v, ci_v, bx_v, by_v, bz_v, bi_v,
                 bcnt_v, gidx_v, w_v, rows_v, out_v, depth_v, sem):
    wid = lax.axis_index("s") * 2 + lax.axis_index("c")
    iota = lax.iota(jnp.int32, 16)
    ones_m = iota < 16
    low8 = iota < 8

    pltpu.sync_copy(pts_hbm, pts_v)

    r0 = wid * ROWS_PER_W
    r0f = r0.astype(jnp.float32)
    y_lo = (r0f + 0.5) * (2.0 / S) - 1.0 - R
    y_hi = (r0f + (ROWS_PER_W - 1) + 0.5) * (2.0 / S) - 1.0 + R

    def scan_pts(i, cnt):
        for h in range(2):
            ridx = i * 32 + h * 16 + iota
            yv = pts_v[pl.ds(P + i * 32 + h * 16, 16)]
            m = (yv >= y_lo) & (yv <= y_hi)
            mi = m.astype(jnp.int32)
            pos = jnp.minimum(cnt + plsc.cumsum(mi) - 1, CAP - 1)
            plsc.store_scatter(ci_v, [pos], ridx, mask=m)
            cnt = jnp.minimum(cnt + jnp.sum(mi), CAP)
        return cnt

    with jax.named_scope("ph_a_compact"):
        cnt = lax.fori_loop(0, PT_CHUNKS // 2, scan_pts, jnp.int32(0))
    n_chunks = (cnt + 15) // 16

    def bin_one(b, _):
        bf = b.astype(jnp.float32)
        xlo = (bf * COLS_PER_B + 0.5) * (2.0 / S) - 1.0 - R
        xhi = (bf * COLS_PER_B + (COLS_PER_B - 1) + 0.5) * (2.0 / S) - 1.0 + R

        def bin_scan(j, bcnt):
            lane = j * 16 + iota
            vl = lane < cnt
            iv = jnp.where(vl, ci_v[pl.ds(j * 16, 16)], 0)
            xv = plsc.load_gather(pts_v, [iv])
            m = vl & (xv >= xlo) & (xv <= xhi)
            mi = m.astype(jnp.int32)
            pos = b * BCAP + jnp.minimum(bcnt + plsc.cumsum(mi) - 1, BCAP - 1)
            yv = plsc.load_gather(pts_v, [iv + P])
            zv = plsc.load_gather(pts_v, [iv + 2 * P])
            plsc.store_scatter(bx_v, [pos], xv, mask=m)
            plsc.store_scatter(by_v, [pos], yv, mask=m)
            plsc.store_scatter(bz_v, [pos], zv, mask=m)
            plsc.store_scatter(bi_v, [pos], iv, mask=m)
            return jnp.minimum(bcnt + jnp.sum(mi), BCAP - 16)

        bcnt = lax.fori_loop(0, n_chunks, bin_scan, jnp.int32(0))
        plsc.store_scatter(bx_v, [b * BCAP + bcnt + iota],
                           jnp.full((16,), BIG, jnp.float32), mask=ones_m)
        plsc.store_scatter(bcnt_v, [jnp.full((16,), b, jnp.int32)],
                           jnp.full((16,), bcnt, jnp.int32), mask=iota == 0)
        return _

    with jax.named_scope("ph_a2_bin"):
        lax.fori_loop(0, NB, bin_one, jnp.int32(0))
    bcv = bcnt_v[pl.ds(0, 16)]

    cyp0 = (r0.astype(jnp.float32) + 0.5) * (2.0 / S) - 1.0
    cyp1 = cyp0 + (2.0 / S)

    def per_col(col, _):
        b = col // COLS_PER_B
        cxp = (col.astype(jnp.float32) + 0.5) * (2.0 / S) - 1.0
        nb_b = jnp.sum(jnp.where(iota == b, bcv, 0))
        nbch = (nb_b + 15) // 16
        bin0 = b * BCAP

        def scan_chunk(j, carry):
            bz0, bp0, bz1, bp1 = carry
            base = bin0 + j * 16
            xv = bx_v[pl.ds(base, 16)]
            yv = by_v[pl.ds(base, 16)]
            zv = bz_v[pl.ds(base, 16)]
            dx = xv - cxp
            dxx = dx * dx
            dy0 = yv - cyp0
            dy1 = yv - cyp1
            d20 = dxx + dy0 * dy0
            d21 = dxx + dy1 * dy1
            zc0 = jnp.where(d20 < R2, zv, BIG)
            zc1 = jnp.where(d21 < R2, zv, BIG)
            pv = base + iota
            zd0, pd0 = plsc.sort_key_val(zc0, pv, descending=True)
            zd1, pd1 = plsc.sort_key_val(zc1, pv, descending=True)
            nz0, np0 = plsc.sort_key_val(jnp.where(low8, bz0, zd0),
                                         jnp.where(low8, bp0, pd0))
            nz1, np1 = plsc.sort_key_val(jnp.where(low8, bz1, zd1),
                                         jnp.where(low8, bp1, pd1))
            return (nz0, np0, nz1, np1)

        big0 = jnp.full((16,), BIG, jnp.float32)
        zero0 = jnp.zeros((16,), jnp.int32)
        bz0, bp0, bz1, bp1 = lax.fori_loop(
            0, nbch, scan_chunk, (big0, zero0, big0, zero0))

        for (bzv, bpv, ps, cyp) in ((bz0, bp0, col, cyp0),
                                    (bz1, bp1, col + S, cyp1)):
            valid = low8 & (bzv < 100.0)
            safe_p = jnp.where(valid, bpv, 0)
            gi = jnp.where(valid, plsc.load_gather(bi_v, [safe_p]), 0)
            gx = plsc.load_gather(bx_v, [safe_p])
            gy = plsc.load_gather(by_v, [safe_p])
            gz = plsc.load_gather(bz_v, [safe_p])
            dx = gx - cxp
            dy = gy - cyp
            d2 = dx * dx + dy * dy
            w = jnp.where(valid, 1.0 - d2 / jnp.float32(R2), 0.0)
            den = jnp.sum(w)
            denv = jnp.maximum(jnp.full((16,), den, jnp.float32), 1e-10)
            w = w / denv

            has0 = jnp.sum(jnp.where(valid & (iota == 0), 1, 0)) > 0
            z0 = jnp.sum(jnp.where(iota == 0, gz, 0.0))
            depth = jnp.where(has0, z0, -1.0)
            rowi = jnp.full((16,), ps // S, jnp.int32)
            coli = jnp.full((16,), col, jnp.int32)
            plsc.store_scatter(depth_v, [rowi, coli],
                               jnp.full((16,), depth, jnp.float32),
                               mask=iota == 0)
            plsc.store_scatter(gidx_v, [ps * K + iota], gi, mask=low8)
            plsc.store_scatter(w_v, [ps * 16 + iota], w, mask=ones_m)
        return _

    def bin_block(b, _):
        lo = b * COLS_PER_B
        lax.fori_loop(lo, lo + COLS_PER_B, per_col, jnp.int32(0))
        for segbase in (0, S * K):
            seg = segbase + b * (COLS_PER_B * K)
            pltpu.async_copy(
                feat_hbm.at[gidx_v.at[pl.ds(seg, COLS_PER_B * K)]],
                rows_v.at[pl.ds(seg, COLS_PER_B * K)], sem)
        return _

    with jax.named_scope("ph_b_topk"):
        lax.fori_loop(0, NB, bin_block, jnp.int32(0))

    for b in range(NB):
        for segbase in (0, S * K):
            seg = segbase + b * (COLS_PER_B * K)
            pltpu.make_async_copy(
                feat_hbm.at[gidx_v.at[pl.ds(seg, COLS_PER_B * K)]],
                rows_v.at[pl.ds(seg, COLS_PER_B * K)], sem).wait()

    def composite(c, _):
        wv0 = w_v[pl.ds(c * 16, 16)]
        wv1 = w_v[pl.ds((S + c) * 16, 16)]
        for cb in range(C // 16):
            acc0 = jnp.zeros((16,), jnp.float32)
            acc1 = jnp.zeros((16,), jnp.float32)
            for k in range(K):
                acc0 = acc0 + wv0[k] * rows_v[c * K + k, pl.ds(cb * 16, 16)]
                acc1 = acc1 + wv1[k] * rows_v[(S + c) * K + k,
                                              pl.ds(cb * 16, 16)]
            out_v[0, c, pl.ds(cb * 16, 16)] = acc0
            out_v[1, c, pl.ds(cb * 16, 16)] = acc1
        return _

    with jax.named_scope("ph_c_composite"):
        lax.fori_loop(0, S, composite, jnp.int32(0))

    pltpu.sync_copy(out_v, img_hbm.at[0, pl.ds(r0, ROWS_PER_W)])
    pltpu.sync_copy(depth_v, depth_hbm.at[pl.ds(r0, ROWS_PER_W)])


@jax.jit
def kernel(points, features):
    mesh = plsc.VectorSubcoreMesh(core_axis_name="c", subcore_axis_name="s")
    run = functools.partial(
        pl.kernel,
        mesh=mesh,
        compiler_params=pltpu.CompilerParams(
            needs_layout_passes=False, use_tc_tiling_on_sc=False),
        out_type=[
            jax.ShapeDtypeStruct((1, S, S, C), jnp.float32),
            jax.ShapeDtypeStruct((S, S), jnp.float32),
        ],
        scratch_types=[
            pltpu.VMEM((P * 3,), jnp.float32),
            pltpu.VMEM((CAP,), jnp.int32),
            pltpu.VMEM((NB * BCAP,), jnp.float32),
            pltpu.VMEM((NB * BCAP,), jnp.float32),
            pltpu.VMEM((NB * BCAP,), jnp.float32),
            pltpu.VMEM((NB * BCAP,), jnp.int32),
            pltpu.VMEM((16,), jnp.int32),
            pltpu.VMEM((PIX_PER_W * K,), jnp.int32),
            pltpu.VMEM((PIX_PER_W * 16,), jnp.float32),
            pltpu.VMEM((PIX_PER_W * K, C), jnp.float32),
            pltpu.VMEM((ROWS_PER_W, S, C), jnp.float32),
            pltpu.VMEM((ROWS_PER_W, S), jnp.float32),
            pltpu.SemaphoreType.DMA,
        ],
    )(_kernel_body)

    images, depth = run(points.T.reshape(-1), features)
    return images, depth[..., None]
